# baseline (speedup 1.0000x reference)
import functools

import jax
import jax.numpy as jnp
from jax import lax
from jax.experimental import pallas as pl
from jax.experimental.pallas import tpu as pltpu
from jax.experimental.pallas import tpu_sc as plsc

K = 8
NC, NS, L = 2, 16, 16
NW = NC * NS


def _rnd_bf16(v):
    u = plsc.bitcast(v, jnp.uint32)
    r = (u + jnp.uint32(0x7FFF) + ((u >> jnp.uint32(16)) & jnp.uint32(1))) \
        & jnp.uint32(0xFFFF0000)
    return plsc.bitcast(r, jnp.float32)


def _insert(carry, v, mv):
    d = carry[:K]
    i = carry[K:]
    lt = [v < dj for dj in d]
    nd = [jnp.where(lt[0], v, d[0])]
    ni = [jnp.where(lt[0], mv, i[0])]
    for j in range(1, K):
        nd.append(jnp.where(lt[j], jnp.where(lt[j - 1], d[j - 1], v), d[j]))
        ni.append(jnp.where(lt[j], jnp.where(lt[j - 1], i[j - 1], mv), i[j]))
    return tuple(nd) + tuple(ni)


def _sc_stage(x, y, yf_flat, d_pad):
    B, N, _ = x.shape
    M = y.shape[2]
    qpt = B * N // NW
    nqv = qpt // L
    tiles_per_b = N // qpt

    mesh = plsc.VectorSubcoreMesh(core_axis_name="c", subcore_axis_name="s",
                                  num_cores=NC, num_subcores=NS)

    @functools.partial(
        pl.kernel,
        out_type=jax.ShapeDtypeStruct((B, N, d_pad), jnp.float32),
        mesh=mesh,
        compiler_params=pltpu.CompilerParams(
            needs_layout_passes=False, use_tc_tiling_on_sc=False),
        scratch_types=[
            pltpu.VMEM((3, M), jnp.float32),
            pltpu.VMEM((4, M), jnp.float32),
            pltpu.VMEM((L, 3), jnp.float32),
            pltpu.VMEM((K * L,), jnp.int32),
            pltpu.VMEM((K * L, d_pad), jnp.float32),
            pltpu.VMEM((L, d_pad), jnp.float32),
            pltpu.SemaphoreType.DMA,
        ],
    )
    def sck(x_hbm, y_hbm, yf_hbm, out_hbm, ybuf, ycv, xbuf, idxv, rows,
            obuf, sem):
        wid = lax.axis_index("s") * NC + lax.axis_index("c")
        b = wid // tiles_per_b
        n_base = (wid % tiles_per_b) * qpt

        pltpu.sync_copy(y_hbm.at[b], ybuf)

        def prep(c, _):
            sl = pl.ds(c * L, L)
            y0 = ybuf[0, sl]
            y1 = ybuf[1, sl]
            y2 = ybuf[2, sl]
            ycv[0, sl] = -2.0 * _rnd_bf16(y0)
            ycv[1, sl] = -2.0 * _rnd_bf16(y1)
            ycv[2, sl] = -2.0 * _rnd_bf16(y2)
            ycv[3, sl] = y0 * y0 + y1 * y1 + y2 * y2
            return 0

        lax.fori_loop(0, M // L, prep, 0)

        inf_v = jnp.full((L,), jnp.inf, jnp.float32)
        zero_i = jnp.zeros((L,), jnp.int32)
        one_i = jnp.ones((L,), jnp.int32)
        lane_i = lax.broadcasted_iota(jnp.int32, (L,), 0)
        row_c = [jnp.full((L,), j, jnp.int32) for j in range(4)]

        def per_qv(qv, _):
            nq = n_base + qv * L
            pltpu.sync_copy(x_hbm.at[b, pl.ds(nq, L), :], xbuf)
            xq = [_rnd_bf16(plsc.load_gather(xbuf, [lane_i, row_c[c]]))
                  for c in range(3)]

            def cand(m, carry):
                mv = carry[0]
                g0 = plsc.load_gather(ycv, [row_c[0], mv])
                g1 = plsc.load_gather(ycv, [row_c[1], mv])
                g2 = plsc.load_gather(ycv, [row_c[2], mv])
                g3 = plsc.load_gather(ycv, [row_c[3], mv])
                v = g3 + ((xq[0] * g0 + xq[1] * g1) + xq[2] * g2)
                hit = jnp.any(v < carry[K])
                tail = lax.cond(hit,
                                lambda: _insert(carry[1:], v, mv),
                                lambda: carry[1:])
                return (mv + one_i,) + tail

            init = (zero_i,) + (inf_v,) * K + (zero_i,) * K
            res = lax.fori_loop(0, M, cand, init)
            bv = jnp.full((L,), b * M, jnp.int32)
            for j in range(K):
                idxv[pl.ds(j * L, L)] = res[1 + K + j] + bv
            pltpu.async_copy(yf_hbm.at[idxv], rows, sem).wait()

            def lane(l, _):
                def chunk(c, _):
                    sl = pl.ds(c * L, L)
                    acc = rows[l, sl]
                    for j in range(1, K):
                        acc = acc + rows[j * L + l, sl]
                    obuf[l, sl] = acc * (1.0 / K)
                    return 0
                lax.fori_loop(0, d_pad // L, chunk, 0)
                return 0

            lax.fori_loop(0, L, lane, 0)
            pltpu.sync_copy(obuf, out_hbm.at[b, pl.ds(nq, L), :])
            return 0

        lax.fori_loop(0, nqv, per_qv, 0)

    return sck(x, y, yf_flat)


def _stage(x, y, f):
    B = y.shape[0]
    M = y.shape[2]
    yt = jnp.transpose(y, (0, 2, 1))
    if f is None:
        yf = yt
    else:
        yf = jnp.concatenate([yt, jnp.transpose(f, (0, 2, 1))], axis=2)
    d = yf.shape[2]
    d_pad = ((d + L - 1) // L) * L
    if d_pad != d:
        yf = jnp.pad(yf, ((0, 0), (0, 0), (0, d_pad - d)))
    yf_flat = yf.reshape(B * M, d_pad)
    means = _sc_stage(x, y, yf_flat, d_pad)
    return means[..., :d]


def kernel(inputs, pc_coords0, pc_coords1, pc_feat1, pc_coords2, pc_feat2,
           pc_coords3, pc_feat3):
    s0 = _stage(inputs, pc_coords0, None)
    s1 = _stage(inputs, pc_coords1, pc_feat1)
    s2 = _stage(inputs, pc_coords2, pc_feat2)
    s3 = _stage(inputs, pc_coords3, pc_feat3)
    return jnp.concatenate([inputs, s0, s1, s2, s3], axis=2)


# speedup vs baseline: 1.3058x; 1.3058x over previous
import functools

import jax
import jax.numpy as jnp
from jax import lax
from jax.experimental import pallas as pl
from jax.experimental.pallas import tpu as pltpu
from jax.experimental.pallas import tpu_sc as plsc

K = 8
NC, NS, L = 2, 16, 16
NW = NC * NS
SEED = 256
CH = 256
FILT_UNROLL = 4


def _rnd_bf16(v):
    u = plsc.bitcast(v, jnp.uint32)
    r = (u + jnp.uint32(0x7FFF) + ((u >> jnp.uint32(16)) & jnp.uint32(1))) \
        & jnp.uint32(0xFFFF0000)
    return plsc.bitcast(r, jnp.float32)


def _insert(carry, v, mv):
    d = carry[:K]
    i = carry[K:]
    lt = [v < dj for dj in d]
    nd = [jnp.where(lt[0], v, d[0])]
    ni = [jnp.where(lt[0], mv, i[0])]
    for j in range(1, K):
        nd.append(jnp.where(lt[j], jnp.where(lt[j - 1], d[j - 1], v), d[j]))
        ni.append(jnp.where(lt[j], jnp.where(lt[j - 1], i[j - 1], mv), i[j]))
    return tuple(nd) + tuple(ni)


def _sc_stage(x, y, yf_flat, d_pad):
    B, N, _ = x.shape
    M = y.shape[2]
    qpt = B * N // NW
    nqv = qpt // L
    tiles_per_b = N // qpt

    mesh = plsc.VectorSubcoreMesh(core_axis_name="c", subcore_axis_name="s",
                                  num_cores=NC, num_subcores=NS)

    @functools.partial(
        pl.kernel,
        out_type=jax.ShapeDtypeStruct((B, N, d_pad), jnp.float32),
        mesh=mesh,
        compiler_params=pltpu.CompilerParams(
            needs_layout_passes=False, use_tc_tiling_on_sc=False),
        scratch_types=[
            pltpu.VMEM((3, M), jnp.float32),
            pltpu.VMEM((4, M), jnp.float32),
            pltpu.VMEM((L, 3), jnp.float32),
            pltpu.VMEM((K * L,), jnp.int32),
            pltpu.VMEM((K * L, d_pad), jnp.float32),
            pltpu.VMEM((L, d_pad), jnp.float32),
            pltpu.VMEM((L * CH,), jnp.float32),
            pltpu.VMEM((L * CH,), jnp.int32),
            pltpu.SemaphoreType.DMA,
        ],
    )
    def sck(x_hbm, y_hbm, yf_hbm, out_hbm, ybuf, ycv, xbuf, idxv, rows,
            obuf, svk, svi, sem):
        wid = lax.axis_index("s") * NC + lax.axis_index("c")
        b = wid // tiles_per_b
        n_base = (wid % tiles_per_b) * qpt

        pltpu.sync_copy(y_hbm.at[b], ybuf)

        def prep(c, _):
            sl = pl.ds(c * L, L)
            y0 = ybuf[0, sl]
            y1 = ybuf[1, sl]
            y2 = ybuf[2, sl]
            ycv[0, sl] = -2.0 * _rnd_bf16(y0)
            ycv[1, sl] = -2.0 * _rnd_bf16(y1)
            ycv[2, sl] = -2.0 * _rnd_bf16(y2)
            ycv[3, sl] = y0 * y0 + y1 * y1 + y2 * y2
            return 0

        lax.fori_loop(0, M // L, prep, 0)

        inf_v = jnp.full((L,), jnp.inf, jnp.float32)
        zero_i = jnp.zeros((L,), jnp.int32)
        one_i = jnp.ones((L,), jnp.int32)
        lane_i = lax.broadcasted_iota(jnp.int32, (L,), 0)
        lane_base = lane_i * CH
        row_c = [jnp.full((L,), j, jnp.int32) for j in range(4)]

        def per_qv(qv, _):
            nq = n_base + qv * L
            pltpu.sync_copy(x_hbm.at[b, pl.ds(nq, L), :], xbuf)
            xq = [_rnd_bf16(plsc.load_gather(xbuf, [lane_i, row_c[c]]))
                  for c in range(3)]

            def key_at(mv):
                g0 = plsc.load_gather(ycv, [row_c[0], mv])
                g1 = plsc.load_gather(ycv, [row_c[1], mv])
                g2 = plsc.load_gather(ycv, [row_c[2], mv])
                g3 = plsc.load_gather(ycv, [row_c[3], mv])
                return g3 + ((xq[0] * g0 + xq[1] * g1) + xq[2] * g2)

            def cand(m, carry):
                mv = carry[0]
                v = key_at(mv)
                return (mv + one_i,) + _insert(carry[1:], v, mv)

            init = (zero_i,) + (inf_v,) * K + (zero_i,) * K
            st = lax.fori_loop(0, SEED, cand, init)

            def chunk_body(ci, st):
                d7 = st[K]

                def filt(j, c2):
                    mv, cnt = c2
                    for _ in range(FILT_UNROLL):
                        v = key_at(mv)
                        msk = v < d7
                        addr = lane_base + cnt
                        plsc.store_scatter(svk, [addr], v, mask=msk)
                        plsc.store_scatter(svi, [addr], mv, mask=msk)
                        cnt = cnt + msk.astype(jnp.int32)
                        mv = mv + one_i
                    return (mv, cnt)

                mv_end, cnt = lax.fori_loop(0, CH // FILT_UNROLL, filt,
                                            (st[0], zero_i))
                maxc = jnp.max(cnt)

                def drain(t, c3):
                    tv = c3[0]
                    addr = lane_base + tv
                    gk = plsc.load_gather(svk, [addr])
                    gi = plsc.load_gather(svi, [addr])
                    vv = jnp.where(tv < cnt, gk, jnp.inf)
                    return (tv + one_i,) + _insert(c3[1:], vv, gi)

                c3 = lax.fori_loop(0, maxc, drain, (zero_i,) + st[1:])
                return (mv_end,) + c3[1:]

            res = lax.fori_loop(0, (M - SEED) // CH, chunk_body, st)
            bv = jnp.full((L,), b * M, jnp.int32)
            for j in range(K):
                idxv[pl.ds(j * L, L)] = res[1 + K + j] + bv
            pltpu.async_copy(yf_hbm.at[idxv], rows, sem).wait()

            def lane(l, _):
                def chunk(c, _):
                    sl = pl.ds(c * L, L)
                    acc = rows[l, sl]
                    for j in range(1, K):
                        acc = acc + rows[j * L + l, sl]
                    obuf[l, sl] = acc * (1.0 / K)
                    return 0
                lax.fori_loop(0, d_pad // L, chunk, 0)
                return 0

            lax.fori_loop(0, L, lane, 0)
            pltpu.sync_copy(obuf, out_hbm.at[b, pl.ds(nq, L), :])
            return 0

        lax.fori_loop(0, nqv, per_qv, 0)

    return sck(x, y, yf_flat)


def _stage(x, y, f):
    B = y.shape[0]
    M = y.shape[2]
    yt = jnp.transpose(y, (0, 2, 1))
    if f is None:
        yf = yt
    else:
        yf = jnp.concatenate([yt, jnp.transpose(f, (0, 2, 1))], axis=2)
    d = yf.shape[2]
    d_pad = ((d + L - 1) // L) * L
    if d_pad != d:
        yf = jnp.pad(yf, ((0, 0), (0, 0), (0, d_pad - d)))
    yf_flat = yf.reshape(B * M, d_pad)
    means = _sc_stage(x, y, yf_flat, d_pad)
    return means[..., :d]


def kernel(inputs, pc_coords0, pc_coords1, pc_feat1, pc_coords2, pc_feat2,
           pc_coords3, pc_feat3):
    s0 = _stage(inputs, pc_coords0, None)
    s1 = _stage(inputs, pc_coords1, pc_feat1)
    s2 = _stage(inputs, pc_coords2, pc_feat2)
    s3 = _stage(inputs, pc_coords3, pc_feat3)
    return jnp.concatenate([inputs, s0, s1, s2, s3], axis=2)


# speedup vs baseline: 1.3322x; 1.0203x over previous
import functools

import jax
import jax.numpy as jnp
from jax import lax
from jax.experimental import pallas as pl
from jax.experimental.pallas import tpu as pltpu
from jax.experimental.pallas import tpu_sc as plsc

K = 8
NC, NS, L = 2, 16, 16
NW = NC * NS
SEED = 256
CH = 256
FILT_UNROLL = 8


def _rnd_bf16(v):
    u = plsc.bitcast(v, jnp.uint32)
    r = (u + jnp.uint32(0x7FFF) + ((u >> jnp.uint32(16)) & jnp.uint32(1))) \
        & jnp.uint32(0xFFFF0000)
    return plsc.bitcast(r, jnp.float32)


def _insert(carry, v, mv):
    d = carry[:K]
    i = carry[K:]
    lt = [v < dj for dj in d]
    nd = [jnp.where(lt[0], v, d[0])]
    ni = [jnp.where(lt[0], mv, i[0])]
    for j in range(1, K):
        nd.append(jnp.where(lt[j], jnp.where(lt[j - 1], d[j - 1], v), d[j]))
        ni.append(jnp.where(lt[j], jnp.where(lt[j - 1], i[j - 1], mv), i[j]))
    return tuple(nd) + tuple(ni)


def _sc_stage(x, y, yf_flat, d_pad):
    B, N, _ = x.shape
    M = y.shape[2]
    qpt = B * N // NW
    nqv = qpt // L
    tiles_per_b = N // qpt

    mesh = plsc.VectorSubcoreMesh(core_axis_name="c", subcore_axis_name="s",
                                  num_cores=NC, num_subcores=NS)

    @functools.partial(
        pl.kernel,
        out_type=jax.ShapeDtypeStruct((B, N, d_pad), jnp.float32),
        mesh=mesh,
        compiler_params=pltpu.CompilerParams(
            needs_layout_passes=False, use_tc_tiling_on_sc=False),
        scratch_types=[
            pltpu.VMEM((3, M), jnp.float32),
            pltpu.VMEM((4, M), jnp.float32),
            pltpu.VMEM((L, 3), jnp.float32),
            pltpu.VMEM((K * L,), jnp.int32),
            pltpu.VMEM((K * L, d_pad), jnp.float32),
            pltpu.VMEM((L, d_pad), jnp.float32),
            pltpu.VMEM((L * CH,), jnp.float32),
            pltpu.VMEM((L * CH,), jnp.int32),
            pltpu.SemaphoreType.DMA,
        ],
    )
    def sck(x_hbm, y_hbm, yf_hbm, out_hbm, ybuf, ycv, xbuf, idxv, rows,
            obuf, svk, svi, sem):
        wid = lax.axis_index("s") * NC + lax.axis_index("c")
        b = wid // tiles_per_b
        n_base = (wid % tiles_per_b) * qpt

        pltpu.sync_copy(y_hbm.at[b], ybuf)

        def prep(c, _):
            sl = pl.ds(c * L, L)
            y0 = ybuf[0, sl]
            y1 = ybuf[1, sl]
            y2 = ybuf[2, sl]
            ycv[0, sl] = -2.0 * _rnd_bf16(y0)
            ycv[1, sl] = -2.0 * _rnd_bf16(y1)
            ycv[2, sl] = -2.0 * _rnd_bf16(y2)
            ycv[3, sl] = y0 * y0 + y1 * y1 + y2 * y2
            return 0

        lax.fori_loop(0, M // L, prep, 0)

        inf_v = jnp.full((L,), jnp.inf, jnp.float32)
        zero_i = jnp.zeros((L,), jnp.int32)
        one_i = jnp.ones((L,), jnp.int32)
        lane_i = lax.broadcasted_iota(jnp.int32, (L,), 0)
        lane_base = lane_i * CH
        row_c = [jnp.full((L,), j, jnp.int32) for j in range(4)]

        def per_qv(qv, _):
            nq = n_base + qv * L
            pltpu.sync_copy(x_hbm.at[b, pl.ds(nq, L), :], xbuf)
            xq = [_rnd_bf16(plsc.load_gather(xbuf, [lane_i, row_c[c]]))
                  for c in range(3)]

            def key_at(mv):
                g0 = plsc.load_gather(ycv, [row_c[0], mv])
                g1 = plsc.load_gather(ycv, [row_c[1], mv])
                g2 = plsc.load_gather(ycv, [row_c[2], mv])
                g3 = plsc.load_gather(ycv, [row_c[3], mv])
                return g3 + ((xq[0] * g0 + xq[1] * g1) + xq[2] * g2)

            def cand(m, carry):
                mv = carry[0]
                v = key_at(mv)
                return (mv + one_i,) + _insert(carry[1:], v, mv)

            init = (zero_i,) + (inf_v,) * K + (zero_i,) * K
            st = lax.fori_loop(0, SEED, cand, init)

            def chunk_body(ci, st):
                d7 = st[K]

                def filt(j, c2):
                    mv, cnt = c2
                    for _ in range(FILT_UNROLL):
                        v = key_at(mv)
                        msk = v < d7
                        addr = lane_base + cnt
                        plsc.store_scatter(svk, [addr], v, mask=msk)
                        plsc.store_scatter(svi, [addr], mv, mask=msk)
                        cnt = cnt + msk.astype(jnp.int32)
                        mv = mv + one_i
                    return (mv, cnt)

                mv_end, cnt = lax.fori_loop(0, CH // FILT_UNROLL, filt,
                                            (st[0], zero_i))
                maxc = jnp.max(cnt)

                def drain(t, c3):
                    tv = c3[0]
                    addr = lane_base + tv
                    gk = plsc.load_gather(svk, [addr])
                    gi = plsc.load_gather(svi, [addr])
                    vv = jnp.where(tv < cnt, gk, jnp.inf)
                    return (tv + one_i,) + _insert(c3[1:], vv, gi)

                c3 = lax.fori_loop(0, maxc, drain, (zero_i,) + st[1:])
                return (mv_end,) + c3[1:]

            res = lax.fori_loop(0, (M - SEED) // CH, chunk_body, st)
            bv = jnp.full((L,), b * M, jnp.int32)
            for j in range(K):
                idxv[pl.ds(j * L, L)] = res[1 + K + j] + bv
            pltpu.async_copy(yf_hbm.at[idxv], rows, sem).wait()

            def lane(l, _):
                def chunk(c, _):
                    sl = pl.ds(c * L, L)
                    acc = rows[l, sl]
                    for j in range(1, K):
                        acc = acc + rows[j * L + l, sl]
                    obuf[l, sl] = acc * (1.0 / K)
                    return 0
                lax.fori_loop(0, d_pad // L, chunk, 0)
                return 0

            lax.fori_loop(0, L, lane, 0)
            pltpu.sync_copy(obuf, out_hbm.at[b, pl.ds(nq, L), :])
            return 0

        lax.fori_loop(0, nqv, per_qv, 0)

    return sck(x, y, yf_flat)


def _stage(x, y, f):
    B = y.shape[0]
    M = y.shape[2]
    yt = jnp.transpose(y, (0, 2, 1))
    if f is None:
        yf = yt
    else:
        yf = jnp.concatenate([yt, jnp.transpose(f, (0, 2, 1))], axis=2)
    d = yf.shape[2]
    d_pad = ((d + L - 1) // L) * L
    if d_pad != d:
        yf = jnp.pad(yf, ((0, 0), (0, 0), (0, d_pad - d)))
    yf_flat = yf.reshape(B * M, d_pad)
    means = _sc_stage(x, y, yf_flat, d_pad)
    return means[..., :d]


def kernel(inputs, pc_coords0, pc_coords1, pc_feat1, pc_coords2, pc_feat2,
           pc_coords3, pc_feat3):
    s0 = _stage(inputs, pc_coords0, None)
    s1 = _stage(inputs, pc_coords1, pc_feat1)
    s2 = _stage(inputs, pc_coords2, pc_feat2)
    s3 = _stage(inputs, pc_coords3, pc_feat3)
    return jnp.concatenate([inputs, s0, s1, s2, s3], axis=2)


# speedup vs baseline: 1.4397x; 1.0806x over previous
import functools

import jax
import jax.numpy as jnp
from jax import lax
from jax.experimental import pallas as pl
from jax.experimental.pallas import tpu as pltpu
from jax.experimental.pallas import tpu_sc as plsc

K = 8
NC, NS, L = 2, 16, 16
NW = NC * NS


def _rnd_bf16(v):
    u = plsc.bitcast(v, jnp.uint32)
    r = (u + jnp.uint32(0x7FFF) + ((u >> jnp.uint32(16)) & jnp.uint32(1))) \
        & jnp.uint32(0xFFFF0000)
    return plsc.bitcast(r, jnp.float32)


def _insert(carry, v, mv):
    d = carry[:K]
    i = carry[K:]
    lt = [v < dj for dj in d]
    nd = [jnp.where(lt[0], v, d[0])]
    ni = [jnp.where(lt[0], mv, i[0])]
    for j in range(1, K):
        nd.append(jnp.where(lt[j], jnp.where(lt[j - 1], d[j - 1], v), d[j]))
        ni.append(jnp.where(lt[j], jnp.where(lt[j - 1], i[j - 1], mv), i[j]))
    return tuple(nd) + tuple(ni)


def _sc_stage(x, y, yf_flat, d_pad):
    B, N, _ = x.shape
    M = y.shape[2]
    qpt = B * N // NW
    nqv = qpt // L
    tiles_per_b = N // qpt

    mesh = plsc.VectorSubcoreMesh(core_axis_name="c", subcore_axis_name="s",
                                  num_cores=NC, num_subcores=NS)

    @functools.partial(
        pl.kernel,
        out_type=jax.ShapeDtypeStruct((B, N, d_pad), jnp.float32),
        mesh=mesh,
        compiler_params=pltpu.CompilerParams(
            needs_layout_passes=False, use_tc_tiling_on_sc=False),
        scratch_types=[
            pltpu.VMEM((3, M), jnp.float32),
            pltpu.VMEM((4, M), jnp.float32),
            pltpu.VMEM((L, 3), jnp.float32),
            pltpu.VMEM((K * L,), jnp.int32),
            pltpu.VMEM((K * L, d_pad), jnp.float32),
            pltpu.VMEM((L, d_pad), jnp.float32),
            pltpu.SemaphoreType.DMA,
        ],
    )
    def sck(x_hbm, y_hbm, yf_hbm, out_hbm, ybuf, ycv, xbuf, idxv, rows,
            obuf, sem):
        wid = lax.axis_index("s") * NC + lax.axis_index("c")
        b = wid // tiles_per_b
        n_base = (wid % tiles_per_b) * qpt

        pltpu.sync_copy(y_hbm.at[b], ybuf)

        def prep(c, _):
            sl = pl.ds(c * L, L)
            y0 = ybuf[0, sl]
            y1 = ybuf[1, sl]
            y2 = ybuf[2, sl]
            ycv[0, sl] = -2.0 * _rnd_bf16(y0)
            ycv[1, sl] = -2.0 * _rnd_bf16(y1)
            ycv[2, sl] = -2.0 * _rnd_bf16(y2)
            ycv[3, sl] = y0 * y0 + y1 * y1 + y2 * y2
            return 0

        lax.fori_loop(0, M // L, prep, 0)

        inf_v = jnp.full((L,), jnp.inf, jnp.float32)
        zero_i = jnp.zeros((L,), jnp.int32)
        one_i = jnp.ones((L,), jnp.int32)
        lane_i = lax.broadcasted_iota(jnp.int32, (L,), 0)
        row_c = [jnp.full((L,), j, jnp.int32) for j in range(4)]

        def per_qv(qv, _):
            nq = n_base + qv * L
            pltpu.sync_copy(x_hbm.at[b, pl.ds(nq, L), :], xbuf)
            xq = [_rnd_bf16(plsc.load_gather(xbuf, [lane_i, row_c[c]]))
                  for c in range(3)]

            def cand(m, carry):
                for _ in range(4):
                    mv = carry[0]
                    g0 = plsc.load_gather(ycv, [row_c[0], mv])
                    g1 = plsc.load_gather(ycv, [row_c[1], mv])
                    g2 = plsc.load_gather(ycv, [row_c[2], mv])
                    g3 = plsc.load_gather(ycv, [row_c[3], mv])
                    v = g3 + ((xq[0] * g0 + xq[1] * g1) + xq[2] * g2)
                    carry = (mv + one_i,) + _insert(carry[1:], v, mv)
                return carry

            init = (zero_i,) + (inf_v,) * K + (zero_i,) * K
            res = lax.fori_loop(0, M // 4, cand, init)
            bv = jnp.full((L,), b * M, jnp.int32)
            for j in range(K):
                idxv[pl.ds(j * L, L)] = res[1 + K + j] + bv
            pltpu.async_copy(yf_hbm.at[idxv], rows, sem).wait()

            def lane(l, _):
                def chunk(c, _):
                    sl = pl.ds(c * L, L)
                    acc = rows[l, sl]
                    for j in range(1, K):
                        acc = acc + rows[j * L + l, sl]
                    obuf[l, sl] = acc * (1.0 / K)
                    return 0
                lax.fori_loop(0, d_pad // L, chunk, 0)
                return 0

            lax.fori_loop(0, L, lane, 0)
            pltpu.sync_copy(obuf, out_hbm.at[b, pl.ds(nq, L), :])
            return 0

        lax.fori_loop(0, nqv, per_qv, 0)

    return sck(x, y, yf_flat)


def _stage(x, y, f):
    B = y.shape[0]
    M = y.shape[2]
    yt = jnp.transpose(y, (0, 2, 1))
    if f is None:
        yf = yt
    else:
        yf = jnp.concatenate([yt, jnp.transpose(f, (0, 2, 1))], axis=2)
    d = yf.shape[2]
    d_pad = ((d + L - 1) // L) * L
    if d_pad != d:
        yf = jnp.pad(yf, ((0, 0), (0, 0), (0, d_pad - d)))
    yf_flat = yf.reshape(B * M, d_pad)
    means = _sc_stage(x, y, yf_flat, d_pad)
    return means[..., :d]


def kernel(inputs, pc_coords0, pc_coords1, pc_feat1, pc_coords2, pc_feat2,
           pc_coords3, pc_feat3):
    s0 = _stage(inputs, pc_coords0, None)
    s1 = _stage(inputs, pc_coords1, pc_feat1)
    s2 = _stage(inputs, pc_coords2, pc_feat2)
    s3 = _stage(inputs, pc_coords3, pc_feat3)
    return jnp.concatenate([inputs, s0, s1, s2, s3], axis=2)


# speedup vs baseline: 1.6874x; 1.1721x over previous
import functools

import jax
import jax.numpy as jnp
from jax import lax
from jax.experimental import pallas as pl
from jax.experimental.pallas import tpu as pltpu
from jax.experimental.pallas import tpu_sc as plsc

K = 8
NC, NS, L = 2, 16, 16
NW = NC * NS


def _rnd_bf16(v):
    u = plsc.bitcast(v, jnp.uint32)
    r = (u + jnp.uint32(0x7FFF) + ((u >> jnp.uint32(16)) & jnp.uint32(1))) \
        & jnp.uint32(0xFFFF0000)
    return plsc.bitcast(r, jnp.float32)


def _insert(carry, v, mv):
    d = carry[:K]
    i = carry[K:]
    lt = [v < dj for dj in d]
    nd = [jnp.where(lt[0], v, d[0])]
    ni = [jnp.where(lt[0], mv, i[0])]
    for j in range(1, K):
        nd.append(jnp.where(lt[j], jnp.where(lt[j - 1], d[j - 1], v), d[j]))
        ni.append(jnp.where(lt[j], jnp.where(lt[j - 1], i[j - 1], mv), i[j]))
    return tuple(nd) + tuple(ni)


def _sc_stage(x, y, yf_flat, d_pad):
    B, N, _ = x.shape
    M = y.shape[2]
    qpt = B * N // NW
    nqv = qpt // L
    tiles_per_b = N // qpt

    mesh = plsc.VectorSubcoreMesh(core_axis_name="c", subcore_axis_name="s",
                                  num_cores=NC, num_subcores=NS)

    @functools.partial(
        pl.kernel,
        out_type=jax.ShapeDtypeStruct((B, N, d_pad), jnp.float32),
        mesh=mesh,
        compiler_params=pltpu.CompilerParams(
            needs_layout_passes=False, use_tc_tiling_on_sc=False),
        scratch_types=[
            pltpu.VMEM((3, M), jnp.float32),
            pltpu.VMEM((3, M), jnp.int32),
            pltpu.VMEM((L, 3), jnp.float32),
            pltpu.VMEM((K * L,), jnp.int32),
            pltpu.VMEM((K * L, d_pad), jnp.float32),
            pltpu.VMEM((L, d_pad), jnp.float32),
            pltpu.SemaphoreType.DMA,
        ],
    )
    def sck(x_hbm, y_hbm, yf_hbm, out_hbm, ybuf, ycv, xbuf, idxv, rows,
            obuf, sem):
        wid = lax.axis_index("s") * NC + lax.axis_index("c")
        b = wid // tiles_per_b
        n_base = (wid % tiles_per_b) * qpt

        pltpu.sync_copy(y_hbm.at[b], ybuf)

        def prep(c, _):
            sl = pl.ds(c * L, L)
            y0 = ybuf[0, sl]
            y1 = ybuf[1, sl]
            y2 = ybuf[2, sl]
            c0 = plsc.bitcast(-2.0 * _rnd_bf16(y0), jnp.uint32)
            c1 = plsc.bitcast(-2.0 * _rnd_bf16(y1), jnp.uint32)
            ycv[0, sl] = plsc.bitcast(c0 | (c1 >> jnp.uint32(16)), jnp.int32)
            ycv[1, sl] = plsc.bitcast(-2.0 * _rnd_bf16(y2), jnp.int32)
            ycv[2, sl] = plsc.bitcast(y0 * y0 + y1 * y1 + y2 * y2, jnp.int32)
            return 0

        lax.fori_loop(0, M // L, prep, 0)

        inf_v = jnp.full((L,), jnp.inf, jnp.float32)
        zero_i = jnp.zeros((L,), jnp.int32)
        one_i = jnp.ones((L,), jnp.int32)
        lane_i = lax.broadcasted_iota(jnp.int32, (L,), 0)
        row_c = [jnp.full((L,), j, jnp.int32) for j in range(4)]

        def per_qv(qv, _):
            nq = n_base + qv * L
            pltpu.sync_copy(x_hbm.at[b, pl.ds(nq, L), :], xbuf)
            xq = [_rnd_bf16(plsc.load_gather(xbuf, [lane_i, row_c[c]]))
                  for c in range(3)]

            def cand(m, carry):
                mv = carry[0]
                gp = plsc.bitcast(plsc.load_gather(ycv, [row_c[0], mv]),
                                  jnp.uint32)
                g2 = plsc.bitcast(plsc.load_gather(ycv, [row_c[1], mv]),
                                  jnp.float32)
                g3 = plsc.bitcast(plsc.load_gather(ycv, [row_c[2], mv]),
                                  jnp.float32)
                g0 = plsc.bitcast(gp & jnp.uint32(0xFFFF0000), jnp.float32)
                g1 = plsc.bitcast(gp << jnp.uint32(16), jnp.float32)
                v = g3 + ((xq[0] * g0 + xq[1] * g1) + xq[2] * g2)
                return (mv + one_i,) + _insert(carry[1:], v, mv)

            init = (zero_i,) + (inf_v,) * K + (zero_i,) * K
            res = lax.fori_loop(0, M, cand, init)
            bv = jnp.full((L,), b * M, jnp.int32)
            for j in range(K):
                idxv[pl.ds(j * L, L)] = res[1 + K + j] + bv
            pltpu.async_copy(yf_hbm.at[idxv], rows, sem).wait()

            def lane(l, _):
                def chunk(c, _):
                    sl = pl.ds(c * L, L)
                    acc = rows[l, sl]
                    for j in range(1, K):
                        acc = acc + rows[j * L + l, sl]
                    obuf[l, sl] = acc * (1.0 / K)
                    return 0
                lax.fori_loop(0, d_pad // L, chunk, 0)
                return 0

            lax.fori_loop(0, L, lane, 0)
            pltpu.sync_copy(obuf, out_hbm.at[b, pl.ds(nq, L), :])
            return 0

        lax.fori_loop(0, nqv, per_qv, 0)

    return sck(x, y, yf_flat)


def _stage(x, y, f):
    B = y.shape[0]
    M = y.shape[2]
    yt = jnp.transpose(y, (0, 2, 1))
    if f is None:
        yf = yt
    else:
        yf = jnp.concatenate([yt, jnp.transpose(f, (0, 2, 1))], axis=2)
    d = yf.shape[2]
    d_pad = ((d + L - 1) // L) * L
    if d_pad != d:
        yf = jnp.pad(yf, ((0, 0), (0, 0), (0, d_pad - d)))
    yf_flat = yf.reshape(B * M, d_pad)
    means = _sc_stage(x, y, yf_flat, d_pad)
    return means[..., :d]


def kernel(inputs, pc_coords0, pc_coords1, pc_feat1, pc_coords2, pc_feat2,
           pc_coords3, pc_feat3):
    s0 = _stage(inputs, pc_coords0, None)
    s1 = _stage(inputs, pc_coords1, pc_feat1)
    s2 = _stage(inputs, pc_coords2, pc_feat2)
    s3 = _stage(inputs, pc_coords3, pc_feat3)
    return jnp.concatenate([inputs, s0, s1, s2, s3], axis=2)


# speedup vs baseline: 1.7688x; 1.0482x over previous
import functools

import jax
import jax.numpy as jnp
from jax import lax
from jax.experimental import pallas as pl
from jax.experimental.pallas import tpu as pltpu
from jax.experimental.pallas import tpu_sc as plsc

K = 8
NC, NS, L = 2, 16, 16
NW = NC * NS


def _rnd_bf16(v):
    u = plsc.bitcast(v, jnp.uint32)
    r = (u + jnp.uint32(0x7FFF) + ((u >> jnp.uint32(16)) & jnp.uint32(1))) \
        & jnp.uint32(0xFFFF0000)
    return plsc.bitcast(r, jnp.float32)


def _insert(carry, v, mv):
    d = carry[:K]
    i = carry[K:]
    lt = [v < dj for dj in d]
    nd = [jnp.where(lt[0], v, d[0])]
    ni = [jnp.where(lt[0], mv, i[0])]
    for j in range(1, K):
        nd.append(jnp.where(lt[j], jnp.where(lt[j - 1], d[j - 1], v), d[j]))
        ni.append(jnp.where(lt[j], jnp.where(lt[j - 1], i[j - 1], mv), i[j]))
    return tuple(nd) + tuple(ni)


def _sc_stage(x, y, yf_flat, d_pad):
    B, N, _ = x.shape
    M = y.shape[2]
    qpt = B * N // NW
    nqv = qpt // L
    tiles_per_b = N // qpt

    mesh = plsc.VectorSubcoreMesh(core_axis_name="c", subcore_axis_name="s",
                                  num_cores=NC, num_subcores=NS)

    @functools.partial(
        pl.kernel,
        out_type=jax.ShapeDtypeStruct((B, N, d_pad), jnp.float32),
        mesh=mesh,
        compiler_params=pltpu.CompilerParams(
            needs_layout_passes=False, use_tc_tiling_on_sc=False),
        scratch_types=[
            pltpu.VMEM((3, M), jnp.float32),
            pltpu.VMEM((4, M), jnp.float32),
            pltpu.VMEM((L, 3), jnp.float32),
            pltpu.VMEM((K * L,), jnp.int32),
            pltpu.VMEM((K * L, d_pad), jnp.float32),
            pltpu.VMEM((L, d_pad), jnp.float32),
            pltpu.SemaphoreType.DMA,
        ],
    )
    def sck(x_hbm, y_hbm, yf_hbm, out_hbm, ybuf, ycv, xbuf, idxv, rows,
            obuf, sem):
        wid = lax.axis_index("s") * NC + lax.axis_index("c")
        b = wid // tiles_per_b
        n_base = (wid % tiles_per_b) * qpt

        pltpu.sync_copy(y_hbm.at[b], ybuf)

        def prep(c, _):
            sl = pl.ds(c * L, L)
            y0 = ybuf[0, sl]
            y1 = ybuf[1, sl]
            y2 = ybuf[2, sl]
            ycv[0, sl] = -2.0 * _rnd_bf16(y0)
            ycv[1, sl] = -2.0 * _rnd_bf16(y1)
            ycv[2, sl] = -2.0 * _rnd_bf16(y2)
            ycv[3, sl] = y0 * y0 + y1 * y1 + y2 * y2
            return 0

        lax.fori_loop(0, M // L, prep, 0)

        inf_v = jnp.full((L,), jnp.inf, jnp.float32)
        zero_i = jnp.zeros((L,), jnp.int32)
        one_i = jnp.ones((L,), jnp.int32)
        lane_i = lax.broadcasted_iota(jnp.int32, (L,), 0)
        row_c = [jnp.full((L,), j, jnp.int32) for j in range(4)]

        def per_qv(qv, _):
            nq = n_base + qv * L
            pltpu.sync_copy(x_hbm.at[b, pl.ds(nq, L), :], xbuf)
            xq = [_rnd_bf16(plsc.load_gather(xbuf, [lane_i, row_c[c]]))
                  for c in range(3)]

            def cand(m, carry):
                mv = carry[0]
                g0 = plsc.load_gather(ycv, [row_c[0], mv])
                g1 = plsc.load_gather(ycv, [row_c[1], mv])
                g2 = plsc.load_gather(ycv, [row_c[2], mv])
                g3 = plsc.load_gather(ycv, [row_c[3], mv])
                v = g3 + ((xq[0] * g0 + xq[1] * g1) + xq[2] * g2)
                return (mv + one_i,) + _insert(carry[1:], v, mv)

            init = (zero_i,) + (inf_v,) * K + (zero_i,) * K
            res = lax.fori_loop(0, M, cand, init)
            bv = jnp.full((L,), b * M, jnp.int32)
            for j in range(K):
                idxv[pl.ds(j * L, L)] = res[1 + K + j] + bv
            pltpu.async_copy(yf_hbm.at[idxv], rows, sem).wait()

            def lane(l, _):
                def chunk(c, _):
                    sl = pl.ds(c * L, L)
                    acc = rows[l, sl]
                    for j in range(1, K):
                        acc = acc + rows[j * L + l, sl]
                    obuf[l, sl] = acc * (1.0 / K)
                    return 0
                lax.fori_loop(0, d_pad // L, chunk, 0)
                return 0

            lax.fori_loop(0, L, lane, 0)
            pltpu.sync_copy(obuf, out_hbm.at[b, pl.ds(nq, L), :])
            return 0

        lax.fori_loop(0, nqv, per_qv, 0)

    return sck(x, y, yf_flat)


def _stage(x, y, f):
    B = y.shape[0]
    M = y.shape[2]
    yt = jnp.transpose(y, (0, 2, 1))
    if f is None:
        yf = yt
    else:
        yf = jnp.concatenate([yt, jnp.transpose(f, (0, 2, 1))], axis=2)
    d = yf.shape[2]
    d_pad = ((d + L - 1) // L) * L
    if d_pad != d:
        yf = jnp.pad(yf, ((0, 0), (0, 0), (0, d_pad - d)))
    yf_flat = yf.reshape(B * M, d_pad)
    means = _sc_stage(x, y, yf_flat, d_pad)
    return means[..., :d]


def kernel(inputs, pc_coords0, pc_coords1, pc_feat1, pc_coords2, pc_feat2,
           pc_coords3, pc_feat3):
    s0 = _stage(inputs, pc_coords0, None)
    s1 = _stage(inputs, pc_coords1, pc_feat1)
    s2 = _stage(inputs, pc_coords2, pc_feat2)
    s3 = _stage(inputs, pc_coords3, pc_feat3)
    return jnp.concatenate([inputs, s0, s1, s2, s3], axis=2)
